# baseline (speedup 1.0000x reference)
import functools

import jax
import jax.numpy as jnp
from jax import lax
from jax.experimental import pallas as pl
from jax.experimental.pallas import tpu as pltpu
from jax.experimental.pallas import tpu_sc as plsc

NN = 50000
NV = 25000
D = 64
NE = 800000
NT = 16
CH = 64
CPT = 782
NE_P = NT * CPT * CH
CW = 16
SROWS = 25088
JUNK = NV
R = 1000
WR = 200


def _iprep_chunk(dstv, srcv, gdi, gsi, li, off):
    def _iprep(q, _):
        sl = pl.ds(q * 16, 16)
        dd = dstv[sl]
        if gdi is not None:
            gdi[sl] = jnp.minimum(dd, NN - 1)
        if srcv is not None:
            gsi[sl] = jnp.minimum(srcv[sl], NN - 1)
        ll = dd - off
        ok = (ll >= 0) & (ll < NV)
        li[sl] = jnp.where(ok, ll, JUNK)
        return 0
    lax.fori_loop(0, CH // 16, _iprep, 0)


def _zero_shared(s, buf, s_sh, rows, width):
    zero16 = jnp.zeros((16,), jnp.float32)

    def _zrow(i, _):
        for q in range(width // 16):
            buf[i, pl.ds(q * 16, 16)] = zero16
        return 0
    lax.fori_loop(0, rows, _zrow, 0)

    def _zchunk(k, _):
        j = s + k * NT

        @pl.when(j < SROWS // rows)
        def _():
            pltpu.sync_copy(buf, s_sh.at[pl.ds(j * rows, rows)])
        return 0
    lax.fori_loop(0, (SROWS // rows + NT - 1) // NT, _zchunk, 0)


def _writeback(s, s_sh, out_hbm, off):
    def _wb(k, _):
        j = s + k * NT

        @pl.when(j < NV // WR)
        def _():
            pltpu.sync_copy(s_sh.at[pl.ds(j * WR, WR)],
                            out_hbm.at[pl.ds(off + j * WR, WR)])
        return 0
    lax.fori_loop(0, (NV // WR + NT - 1) // NT, _wb, 0)


def _sc_edge_body(pd_hbm, ps_hbm, src_hbm, dst_hbm, et0_hbm, et1_hbm,
                  w0_hbm, w1_hbm, out_hbm,
                  dstv0, srcv0, gdi0, gsi0, li0, et0v0, et1v0, gdv0, gsv0,
                  dstv1, srcv1, gdi1, gsi1, li1, et0v1, et1v1, gdv1, gsv1,
                  w0v, w1v, s_sh, sem_in0, sem_in1, sem_sc0, sem_sc1):
    c = lax.axis_index("c")
    s = lax.axis_index("s")
    off = c * NV

    pltpu.sync_copy(w0_hbm, w0v)
    pltpu.sync_copy(w1_hbm, w1v)

    _zero_shared(s, gdv0, s_sh, CH, D)
    plsc.subcore_barrier()

    ebase = s * (CPT * CH)
    bufs = ((dstv0, srcv0, gdi0, gsi0, li0, et0v0, et1v0, gdv0, gsv0,
             sem_in0, sem_sc0),
            (dstv1, srcv1, gdi1, gsi1, li1, et0v1, et1v1, gdv1, gsv1,
             sem_in1, sem_sc1))

    def _prep(kn, buf):
        dstv, srcv, gdi, gsi, li, et0v, et1v, gdv, gsv, sem_in, _ = buf
        b = ebase + kn * CH
        pltpu.sync_copy(dst_hbm.at[pl.ds(b, CH)], dstv)
        pltpu.sync_copy(src_hbm.at[pl.ds(b, CH)], srcv)
        _iprep_chunk(dstv, srcv, gdi, gsi, li, off)
        pltpu.async_copy(et0_hbm.at[pl.ds(b, CH)], et0v, sem_in)
        pltpu.async_copy(et1_hbm.at[pl.ds(b, CH)], et1v, sem_in)
        pltpu.async_copy(pd_hbm.at[gdi], gdv, sem_in)
        pltpu.async_copy(ps_hbm.at[gsi], gsv, sem_in)

    def _wait_in(kn, buf):
        dstv, srcv, gdi, gsi, li, et0v, et1v, gdv, gsv, sem_in, _ = buf
        b = ebase + kn * CH
        pltpu.make_async_copy(et0_hbm.at[pl.ds(b, CH)], et0v, sem_in).wait()
        pltpu.make_async_copy(et1_hbm.at[pl.ds(b, CH)], et1v, sem_in).wait()
        pltpu.make_async_copy(pd_hbm.at[gdi], gdv, sem_in).wait()
        pltpu.make_async_copy(ps_hbm.at[gsi], gsv, sem_in).wait()

    def _compute(buf):
        et0v, et1v, gdv, gsv = buf[5], buf[6], buf[7], buf[8]
        w0s = [w0v[pl.ds(j * 16, 16)] for j in range(4)]
        w1s = [w1v[pl.ds(j * 16, 16)] for j in range(4)]

        def _grp(g, _):
            e0g = et0v[pl.ds(g * 16, 16)]
            e1g = et1v[pl.ds(g * 16, 16)]
            base = g * 16
            for l in range(16):
                i = base + l
                e0 = e0g[l]
                e1 = e1g[l]
                for j in range(4):
                    sl = pl.ds(j * 16, 16)
                    m = (gdv[i, sl] + gsv[i, sl]
                         + e0 * w0s[j] + e1 * w1s[j])
                    gdv[i, sl] = jnp.maximum(m, 0.0)
            return 0
        lax.fori_loop(0, CH // 16, _grp, 0)

    def _fire_scatter(buf):
        li, gdv, sem_sc = buf[4], buf[7], buf[10]
        pltpu.async_copy(gdv, s_sh.at[li], sem_sc, add=True)

    def _wait_scatter(buf):
        li, gdv, sem_sc = buf[4], buf[7], buf[10]
        pltpu.make_async_copy(gdv, s_sh.at[li], sem_sc).wait()

    _prep(0, bufs[0])

    def _body(k2, _):
        k0 = 2 * k2

        @pl.when(k2 > 0)
        def _():
            _wait_scatter(bufs[1])
        _prep(k0 + 1, bufs[1])
        _wait_in(k0, bufs[0])
        _compute(bufs[0])
        _fire_scatter(bufs[0])
        _wait_scatter(bufs[0])

        @pl.when(k0 + 2 < CPT)
        def _():
            _prep(k0 + 2, bufs[0])
        _wait_in(k0 + 1, bufs[1])
        _compute(bufs[1])
        _fire_scatter(bufs[1])
        return 0
    lax.fori_loop(0, CPT // 2, _body, 0)
    _wait_scatter(bufs[1])
    plsc.subcore_barrier()
    _writeback(s, s_sh, out_hbm, off)


def _sc_cnt_body(dst_hbm, out_hbm, dstv0, li0, dstv1, li1, onev, s_sh,
                 sem_d0, sem_d1, sem_sc0, sem_sc1):
    c = lax.axis_index("c")
    s = lax.axis_index("s")
    off = c * NV

    _zero_shared(s, onev, s_sh, CH, CW)
    plsc.subcore_barrier()

    lane = lax.iota(jnp.int32, 16)
    onecol = jnp.where(lane == 0, 1.0, 0.0).astype(jnp.float32)

    def _orow(i, _):
        onev[i, pl.ds(0, 16)] = onecol
        return 0
    lax.fori_loop(0, CH, _orow, 0)

    ebase = s * (CPT * CH)
    bufs = ((dstv0, li0, sem_d0, sem_sc0), (dstv1, li1, sem_d1, sem_sc1))

    def _stage(kn, buf):
        dstv, li, sem_d, _ = buf
        pltpu.async_copy(dst_hbm.at[pl.ds(ebase + kn * CH, CH)], dstv, sem_d)

    def _scat(kn, buf):
        dstv, li, sem_d, sem_sc = buf
        pltpu.make_async_copy(dst_hbm.at[pl.ds(ebase + kn * CH, CH)],
                             dstv, sem_d).wait()
        _iprep_chunk(dstv, None, None, None, li, off)
        pltpu.async_copy(onev, s_sh.at[li], sem_sc, add=True)

    def _wait_sc(buf):
        dstv, li, sem_d, sem_sc = buf
        pltpu.make_async_copy(onev, s_sh.at[li], sem_sc).wait()

    _stage(0, bufs[0])

    def _body(k2, _):
        k0 = 2 * k2
        _stage(k0 + 1, bufs[1])

        @pl.when(k2 > 0)
        def _():
            _wait_sc(bufs[0])
        _scat(k0, bufs[0])

        @pl.when(k0 + 2 < CPT)
        def _():
            _stage(k0 + 2, bufs[0])

        @pl.when(k2 > 0)
        def _():
            _wait_sc(bufs[1])
        _scat(k0 + 1, bufs[1])
        return 0
    lax.fori_loop(0, CPT // 2, _body, 0)
    _wait_sc(bufs[0])
    _wait_sc(bufs[1])
    plsc.subcore_barrier()
    _writeback(s, s_sh, out_hbm, off)


_SC_MESH = dict(core_axis_name="c", subcore_axis_name="s",
                num_cores=2, num_subcores=NT)


@functools.cache
def _sc_edge():
    return pl.kernel(
        _sc_edge_body,
        out_type=jax.ShapeDtypeStruct((NN, D), jnp.float32),
        mesh=plsc.VectorSubcoreMesh(**_SC_MESH),
        scratch_types=(
            [pltpu.VMEM((CH,), jnp.int32)] * 5
            + [pltpu.VMEM((CH,), jnp.float32)] * 2
            + [pltpu.VMEM((CH, D), jnp.float32)] * 2
        ) * 2 + [
            pltpu.VMEM((D,), jnp.float32),
            pltpu.VMEM((D,), jnp.float32),
            pltpu.VMEM_SHARED((SROWS, D), jnp.float32),
            pltpu.SemaphoreType.DMA,
            pltpu.SemaphoreType.DMA,
            pltpu.SemaphoreType.DMA,
            pltpu.SemaphoreType.DMA,
        ],
        compiler_params=pltpu.CompilerParams(use_tc_tiling_on_sc=False),
    )


@functools.cache
def _sc_cnt():
    return pl.kernel(
        _sc_cnt_body,
        out_type=jax.ShapeDtypeStruct((NN, CW), jnp.float32),
        mesh=plsc.VectorSubcoreMesh(**_SC_MESH),
        scratch_types=[
            pltpu.VMEM((CH,), jnp.int32),
            pltpu.VMEM((CH,), jnp.int32),
            pltpu.VMEM((CH,), jnp.int32),
            pltpu.VMEM((CH,), jnp.int32),
            pltpu.VMEM((CH, CW), jnp.float32),
            pltpu.VMEM_SHARED((SROWS, CW), jnp.float32),
            pltpu.SemaphoreType.DMA,
            pltpu.SemaphoreType.DMA,
            pltpu.SemaphoreType.DMA,
            pltpu.SemaphoreType.DMA,
        ],
        compiler_params=pltpu.CompilerParams(use_tc_tiling_on_sc=False,
                                             needs_layout_passes=False),
    )


_PERM = sum(([q * 32 + 2 * k for k in range(16)]
             + [q * 32 + 2 * k + 1 for k in range(16)]
             for q in range(2)), [])


def _pre_body(vf, cf, vW1, vb1, vW2, vb2, cW1, cb1, cW2, cb2,
              W1d, nb1, W1s, x0o, pdo, pso):
    isv = pl.program_id(0) < NV // R
    f = jnp.where(isv, vf[...], cf[...])
    W1 = jnp.where(isv, vW1[...], cW1[...])
    b1 = jnp.where(isv, vb1[...], cb1[...])
    W2 = jnp.where(isv, vW2[...], cW2[...])
    b2 = jnp.where(isv, vb2[...], cb2[...])
    x = jax.nn.relu(f @ W1.T + b1) @ W2.T + b2
    x0o[...] = x
    pdo[...] = x @ W1d[...].T + nb1[...]
    pso[...] = x @ W1s[...].T


def _mid_body(S, C, W2, b2, W1d, nb1, W1s, xo, pdo, pso):
    cnt = C[...][:, :1]
    x = jax.nn.relu((S[...] @ W2[...].T + cnt * b2[...])
                    / jnp.maximum(cnt, 1.0))
    xo[...] = x
    pdo[...] = x @ W1d[...].T + nb1[...]
    pso[...] = x @ W1s[...].T


def _fin_body(S3, C, x0, x1, x2, W2, b2, F0, F1, F2, F3,
              fb1, fW2, fb2, fW3, fb3, out):
    cnt = C[...][:, :1]
    x3 = jax.nn.relu((S3[...] @ W2[...].T + cnt * b2[...])
                     / jnp.maximum(cnt, 1.0))
    h = (x0[...] @ F0[...].T + x1[...] @ F1[...].T + x2[...] @ F2[...].T
         + x3 @ F3[...].T + fb1[...])
    h = jax.nn.relu(h)
    h = jax.nn.relu(h @ fW2[...].T + fb2[...])
    out[...] = jax.nn.sigmoid(h @ fW3[...].T + fb3[...])


def _full(shape):
    return pl.BlockSpec(shape, lambda i: tuple(0 for _ in shape))


def _rows(w):
    return pl.BlockSpec((R, w), lambda i: (i, 0))


_pre_call = pl.pallas_call(
    _pre_body,
    grid=(NN // R,),
    in_specs=[
        pl.BlockSpec((R, 2), lambda i: (jnp.minimum(i, NV // R - 1), 0)),
        pl.BlockSpec((R, 2), lambda i: (jnp.maximum(i - NV // R, 0), 0)),
        _full((D, 2)), _full((1, D)), _full((D, D)), _full((1, D)),
        _full((D, 2)), _full((1, D)), _full((D, D)), _full((1, D)),
        _full((D, D)), _full((1, D)), _full((D, D)),
    ],
    out_specs=[_rows(D), _rows(D), _rows(D)],
    out_shape=[jax.ShapeDtypeStruct((NN, D), jnp.float32)] * 3,
)

_mid_call = pl.pallas_call(
    _mid_body,
    grid=(NN // R,),
    in_specs=[
        _rows(D), _rows(CW),
        _full((D, D)), _full((1, D)),
        _full((D, D)), _full((1, D)), _full((D, D)),
    ],
    out_specs=[_rows(D), _rows(D), _rows(D)],
    out_shape=[jax.ShapeDtypeStruct((NN, D), jnp.float32)] * 3,
)

_fin_call = pl.pallas_call(
    _fin_body,
    grid=(NV // R,),
    in_specs=[
        _rows(D), _rows(CW), _rows(D), _rows(D), _rows(D),
        _full((D, D)), _full((1, D)),
        _full((D, D)), _full((D, D)), _full((D, D)), _full((D, D)),
        _full((1, D)), _full((D, D)), _full((1, D)),
        _full((128, D)), _full((1, 128)),
    ],
    out_specs=pl.BlockSpec((R, 128), lambda i: (i, 0)),
    out_shape=jax.ShapeDtypeStruct((NV, 128), jnp.float32),
)


def kernel(var_node_features, con_node_features, node_types, assoc_var,
           assoc_con, edge_index, edge_types, vm_W1, vm_b1, vm_W2, vm_b2,
           cm_W1, cm_b1, cm_W2, cm_b2, c1_W1, c1_b1, c1_W2, c1_b2, c2_W1,
           c2_b1, c2_W2, c2_b2, c3_W1, c3_b1, c3_W2, c3_b2, fc1_W, fc1_b,
           fc2_W, fc2_b, fc3_W, fc3_b):
    f32 = jnp.float32
    r1 = lambda b: b.reshape(1, -1)

    x0, pd, ps = _pre_call(
        var_node_features, con_node_features,
        vm_W1, r1(vm_b1), vm_W2, r1(vm_b2),
        cm_W1, r1(cm_b1), cm_W2, r1(cm_b2),
        c1_W1[:, :D], r1(c1_b1), c1_W1[:, D:2 * D])

    pad = NE_P - NE
    srcp = jnp.concatenate([edge_index[0],
                            jnp.full((pad, ), NN, jnp.int32)])
    dstp = jnp.concatenate([edge_index[1],
                            jnp.full((pad, ), NN, jnp.int32)])
    zpad = jnp.zeros((pad,), f32)
    et0p = jnp.concatenate([edge_types[:, 0], zpad])
    et1p = jnp.concatenate([edge_types[:, 1], zpad])

    cntv = _sc_cnt()(dstp)
    sc = _sc_edge()
    s1 = sc(pd, ps, srcp, dstp, et0p, et1p,
            c1_W1[:, 2 * D], c1_W1[:, 2 * D + 1])
    x1, pd, ps = _mid_call(s1, cntv, c1_W2, r1(c1_b2),
                           c2_W1[:, :D], r1(c2_b1), c2_W1[:, D:2 * D])
    s2 = sc(pd, ps, srcp, dstp, et0p, et1p,
            c2_W1[:, 2 * D], c2_W1[:, 2 * D + 1])
    x2, pd, ps = _mid_call(s2, cntv, c2_W2, r1(c2_b2),
                           c3_W1[:, :D], r1(c3_b1), c3_W1[:, D:2 * D])
    s3 = sc(pd, ps, srcp, dstp, et0p, et1p,
            c3_W1[:, 2 * D], c3_W1[:, 2 * D + 1])

    out = _fin_call(s3, cntv, x0, x1, x2, c3_W2, r1(c3_b2),
                    fc1_W[:, :D], fc1_W[:, D:2 * D],
                    fc1_W[:, 2 * D:3 * D], fc1_W[:, 3 * D:],
                    r1(fc1_b), fc2_W, r1(fc2_b),
                    jnp.zeros((128, D), f32).at[0].set(fc3_W[0]),
                    jnp.zeros((1, 128), f32).at[0, 0].set(fc3_b[0]))
    return out[:, 0]


# speedup vs baseline: 1.4257x; 1.4257x over previous
import functools

import jax
import jax.numpy as jnp
from jax import lax
from jax.experimental import pallas as pl
from jax.experimental.pallas import tpu as pltpu
from jax.experimental.pallas import tpu_sc as plsc

NN = 50000
NV = 25000
D = 64
NE = 800000
NT = 16
CH = 64
CPT = 782
NE_P = NT * CPT * CH
CW = 16
SROWS = 25088
JUNK = NV
R = 1000
WR = 200


def _iprep_chunk(dstv, srcv, gdi, gsi, li, off):
    def _iprep(q, _):
        sl = pl.ds(q * 16, 16)
        dd = dstv[sl]
        if gdi is not None:
            gdi[sl] = jnp.minimum(dd, NN - 1)
        if srcv is not None:
            gsi[sl] = jnp.minimum(srcv[sl], NN - 1)
        ll = dd - off
        ok = (ll >= 0) & (ll < NV)
        li[sl] = jnp.where(ok, ll, JUNK)
        return 0
    lax.fori_loop(0, CH // 16, _iprep, 0)


def _zero_shared(s, buf, s_sh, rows, width):
    zero16 = jnp.zeros((16,), jnp.float32)

    def _zrow(i, _):
        for q in range(width // 16):
            buf[i, pl.ds(q * 16, 16)] = zero16
        return 0
    lax.fori_loop(0, rows, _zrow, 0)

    def _zchunk(k, _):
        j = s + k * NT

        @pl.when(j < SROWS // rows)
        def _():
            pltpu.sync_copy(buf, s_sh.at[pl.ds(j * rows, rows)])
        return 0
    lax.fori_loop(0, (SROWS // rows + NT - 1) // NT, _zchunk, 0)


def _writeback(s, s_sh, out_hbm, off):
    def _wb(k, _):
        j = s + k * NT

        @pl.when(j < NV // WR)
        def _():
            pltpu.sync_copy(s_sh.at[pl.ds(j * WR, WR)],
                            out_hbm.at[pl.ds(off + j * WR, WR)])
        return 0
    lax.fori_loop(0, (NV // WR + NT - 1) // NT, _wb, 0)


def _sc_edge_body(pd_hbm, ps_hbm, src_hbm, dst_hbm, pe_hbm, out_hbm,
                  dstv0, srcv0, gdi0, gsi0, li0, pev0, gdv0, gsv0,
                  dstv1, srcv1, gdi1, gsi1, li1, pev1, gdv1, gsv1,
                  s_sh, sem_in0, sem_in1, sem_sc0, sem_sc1):
    c = lax.axis_index("c")
    s = lax.axis_index("s")
    off = c * NV

    _zero_shared(s, gdv0, s_sh, CH, D)
    plsc.subcore_barrier()

    ebase = s * (CPT * CH)
    bufs = ((dstv0, srcv0, gdi0, gsi0, li0, pev0, gdv0, gsv0,
             sem_in0, sem_sc0),
            (dstv1, srcv1, gdi1, gsi1, li1, pev1, gdv1, gsv1,
             sem_in1, sem_sc1))

    def _prep(kn, buf):
        dstv, srcv, gdi, gsi, li, pev, gdv, gsv, sem_in, _ = buf
        b = ebase + kn * CH
        pltpu.sync_copy(dst_hbm.at[pl.ds(b, CH)], dstv)
        pltpu.sync_copy(src_hbm.at[pl.ds(b, CH)], srcv)
        _iprep_chunk(dstv, srcv, gdi, gsi, li, off)
        pltpu.async_copy(pe_hbm.at[pl.ds(b // 2, CH // 2)], pev, sem_in)
        pltpu.async_copy(pd_hbm.at[gdi], gdv, sem_in)
        pltpu.async_copy(ps_hbm.at[gsi], gsv, sem_in)

    def _wait_in(kn, buf):
        dstv, srcv, gdi, gsi, li, pev, gdv, gsv, sem_in, _ = buf
        b = ebase + kn * CH
        pltpu.make_async_copy(pe_hbm.at[pl.ds(b // 2, CH // 2)],
                              pev, sem_in).wait()
        pltpu.make_async_copy(pd_hbm.at[gdi], gdv, sem_in).wait()
        pltpu.make_async_copy(ps_hbm.at[gsi], gsv, sem_in).wait()

    def _compute(buf):
        pev, gdv, gsv = buf[5], buf[6], buf[7]

        def _pair(i2, _):
            for h in range(2):
                i = 2 * i2 + h
                for j in range(4):
                    sl = pl.ds(j * 16, 16)
                    m = (gdv[i, sl] + gsv[i, sl]
                         + pev[i2, pl.ds(h * 64 + j * 16, 16)])
                    gdv[i, sl] = jnp.maximum(m, 0.0)
            return 0
        lax.fori_loop(0, CH // 2, _pair, 0)

    def _fire_scatter(buf):
        li, gdv, sem_sc = buf[4], buf[6], buf[9]
        pltpu.async_copy(gdv, s_sh.at[li], sem_sc, add=True)

    def _wait_scatter(buf):
        li, gdv, sem_sc = buf[4], buf[6], buf[9]
        pltpu.make_async_copy(gdv, s_sh.at[li], sem_sc).wait()

    _prep(0, bufs[0])

    def _body(k2, _):
        k0 = 2 * k2

        @pl.when(k2 > 0)
        def _():
            _wait_scatter(bufs[1])
        _prep(k0 + 1, bufs[1])
        _wait_in(k0, bufs[0])
        _compute(bufs[0])
        _fire_scatter(bufs[0])
        _wait_scatter(bufs[0])

        @pl.when(k0 + 2 < CPT)
        def _():
            _prep(k0 + 2, bufs[0])
        _wait_in(k0 + 1, bufs[1])
        _compute(bufs[1])
        _fire_scatter(bufs[1])
        return 0
    lax.fori_loop(0, CPT // 2, _body, 0)
    _wait_scatter(bufs[1])
    plsc.subcore_barrier()
    _writeback(s, s_sh, out_hbm, off)


def _sc_cnt_body(dst_hbm, out_hbm, dstv0, li0, dstv1, li1, onev, s_sh,
                 sem_d0, sem_d1, sem_sc0, sem_sc1):
    c = lax.axis_index("c")
    s = lax.axis_index("s")
    off = c * NV

    _zero_shared(s, onev, s_sh, CH, CW)
    plsc.subcore_barrier()

    lane = lax.iota(jnp.int32, 16)
    onecol = jnp.where(lane == 0, 1.0, 0.0).astype(jnp.float32)

    def _orow(i, _):
        onev[i, pl.ds(0, 16)] = onecol
        return 0
    lax.fori_loop(0, CH, _orow, 0)

    ebase = s * (CPT * CH)
    bufs = ((dstv0, li0, sem_d0, sem_sc0), (dstv1, li1, sem_d1, sem_sc1))

    def _stage(kn, buf):
        dstv, li, sem_d, _ = buf
        pltpu.async_copy(dst_hbm.at[pl.ds(ebase + kn * CH, CH)], dstv, sem_d)

    def _scat(kn, buf):
        dstv, li, sem_d, sem_sc = buf
        pltpu.make_async_copy(dst_hbm.at[pl.ds(ebase + kn * CH, CH)],
                             dstv, sem_d).wait()
        _iprep_chunk(dstv, None, None, None, li, off)
        pltpu.async_copy(onev, s_sh.at[li], sem_sc, add=True)

    def _wait_sc(buf):
        dstv, li, sem_d, sem_sc = buf
        pltpu.make_async_copy(onev, s_sh.at[li], sem_sc).wait()

    _stage(0, bufs[0])

    def _body(k2, _):
        k0 = 2 * k2
        _stage(k0 + 1, bufs[1])

        @pl.when(k2 > 0)
        def _():
            _wait_sc(bufs[0])
        _scat(k0, bufs[0])

        @pl.when(k0 + 2 < CPT)
        def _():
            _stage(k0 + 2, bufs[0])

        @pl.when(k2 > 0)
        def _():
            _wait_sc(bufs[1])
        _scat(k0 + 1, bufs[1])
        return 0
    lax.fori_loop(0, CPT // 2, _body, 0)
    _wait_sc(bufs[0])
    _wait_sc(bufs[1])
    plsc.subcore_barrier()
    _writeback(s, s_sh, out_hbm, off)


_SC_MESH = dict(core_axis_name="c", subcore_axis_name="s",
                num_cores=2, num_subcores=NT)


@functools.cache
def _sc_edge():
    return pl.kernel(
        _sc_edge_body,
        out_type=jax.ShapeDtypeStruct((NN, D), jnp.float32),
        mesh=plsc.VectorSubcoreMesh(**_SC_MESH),
        scratch_types=(
            [pltpu.VMEM((CH,), jnp.int32)] * 5
            + [pltpu.VMEM((CH // 2, 2 * D), jnp.float32),
               pltpu.VMEM((CH, D), jnp.float32),
               pltpu.VMEM((CH, D), jnp.float32)]
        ) * 2 + [
            pltpu.VMEM_SHARED((SROWS, D), jnp.float32),
            pltpu.SemaphoreType.DMA,
            pltpu.SemaphoreType.DMA,
            pltpu.SemaphoreType.DMA,
            pltpu.SemaphoreType.DMA,
        ],
        compiler_params=pltpu.CompilerParams(use_tc_tiling_on_sc=False),
    )


@functools.cache
def _sc_cnt():
    return pl.kernel(
        _sc_cnt_body,
        out_type=jax.ShapeDtypeStruct((NN, CW), jnp.float32),
        mesh=plsc.VectorSubcoreMesh(**_SC_MESH),
        scratch_types=[
            pltpu.VMEM((CH,), jnp.int32),
            pltpu.VMEM((CH,), jnp.int32),
            pltpu.VMEM((CH,), jnp.int32),
            pltpu.VMEM((CH,), jnp.int32),
            pltpu.VMEM((CH, CW), jnp.float32),
            pltpu.VMEM_SHARED((SROWS, CW), jnp.float32),
            pltpu.SemaphoreType.DMA,
            pltpu.SemaphoreType.DMA,
            pltpu.SemaphoreType.DMA,
            pltpu.SemaphoreType.DMA,
        ],
        compiler_params=pltpu.CompilerParams(use_tc_tiling_on_sc=False,
                                             needs_layout_passes=False),
    )


_PERM = sum(([q * 32 + 2 * k for k in range(16)]
             + [q * 32 + 2 * k + 1 for k in range(16)]
             for q in range(2)), [])


def _pe_body(et, w, peo):
    peo[...] = et[...] @ w[...].T


RE = 25024

_pe_call = pl.pallas_call(
    _pe_body,
    grid=(NE_P // 2 // RE,),
    in_specs=[pl.BlockSpec((RE, 4), lambda i: (i, 0)),
              pl.BlockSpec((2 * D, 4), lambda i: (0, 0))],
    out_specs=pl.BlockSpec((RE, 2 * D), lambda i: (i, 0)),
    out_shape=jax.ShapeDtypeStruct((NE_P // 2, 2 * D), jnp.float32),
)


def _pre_body(vf, cf, vW1, vb1, vW2, vb2, cW1, cb1, cW2, cb2,
              W1d, nb1, W1s, x0o, pdo, pso):
    isv = pl.program_id(0) < NV // R
    f = jnp.where(isv, vf[...], cf[...])
    W1 = jnp.where(isv, vW1[...], cW1[...])
    b1 = jnp.where(isv, vb1[...], cb1[...])
    W2 = jnp.where(isv, vW2[...], cW2[...])
    b2 = jnp.where(isv, vb2[...], cb2[...])
    x = jax.nn.relu(f @ W1.T + b1) @ W2.T + b2
    x0o[...] = x
    pdo[...] = x @ W1d[...].T + nb1[...]
    pso[...] = x @ W1s[...].T


def _mid_body(S, C, W2, b2, W1d, nb1, W1s, xo, pdo, pso):
    cnt = C[...][:, :1]
    x = jax.nn.relu((S[...] @ W2[...].T + cnt * b2[...])
                    / jnp.maximum(cnt, 1.0))
    xo[...] = x
    pdo[...] = x @ W1d[...].T + nb1[...]
    pso[...] = x @ W1s[...].T


def _fin_body(S3, C, x0, x1, x2, W2, b2, F0, F1, F2, F3,
              fb1, fW2, fb2, fW3, fb3, out):
    cnt = C[...][:, :1]
    x3 = jax.nn.relu((S3[...] @ W2[...].T + cnt * b2[...])
                     / jnp.maximum(cnt, 1.0))
    h = (x0[...] @ F0[...].T + x1[...] @ F1[...].T + x2[...] @ F2[...].T
         + x3 @ F3[...].T + fb1[...])
    h = jax.nn.relu(h)
    h = jax.nn.relu(h @ fW2[...].T + fb2[...])
    out[...] = jax.nn.sigmoid(h @ fW3[...].T + fb3[...])


def _full(shape):
    return pl.BlockSpec(shape, lambda i: tuple(0 for _ in shape))


def _rows(w):
    return pl.BlockSpec((R, w), lambda i: (i, 0))


_pre_call = pl.pallas_call(
    _pre_body,
    grid=(NN // R,),
    in_specs=[
        pl.BlockSpec((R, 2), lambda i: (jnp.minimum(i, NV // R - 1), 0)),
        pl.BlockSpec((R, 2), lambda i: (jnp.maximum(i - NV // R, 0), 0)),
        _full((D, 2)), _full((1, D)), _full((D, D)), _full((1, D)),
        _full((D, 2)), _full((1, D)), _full((D, D)), _full((1, D)),
        _full((D, D)), _full((1, D)), _full((D, D)),
    ],
    out_specs=[_rows(D), _rows(D), _rows(D)],
    out_shape=[jax.ShapeDtypeStruct((NN, D), jnp.float32)] * 3,
)

_mid_call = pl.pallas_call(
    _mid_body,
    grid=(NN // R,),
    in_specs=[
        _rows(D), _rows(CW),
        _full((D, D)), _full((1, D)),
        _full((D, D)), _full((1, D)), _full((D, D)),
    ],
    out_specs=[_rows(D), _rows(D), _rows(D)],
    out_shape=[jax.ShapeDtypeStruct((NN, D), jnp.float32)] * 3,
)

_fin_call = pl.pallas_call(
    _fin_body,
    grid=(NV // R,),
    in_specs=[
        _rows(D), _rows(CW), _rows(D), _rows(D), _rows(D),
        _full((D, D)), _full((1, D)),
        _full((D, D)), _full((D, D)), _full((D, D)), _full((D, D)),
        _full((1, D)), _full((D, D)), _full((1, D)),
        _full((128, D)), _full((1, 128)),
    ],
    out_specs=pl.BlockSpec((R, 128), lambda i: (i, 0)),
    out_shape=jax.ShapeDtypeStruct((NV, 128), jnp.float32),
)


def kernel(var_node_features, con_node_features, node_types, assoc_var,
           assoc_con, edge_index, edge_types, vm_W1, vm_b1, vm_W2, vm_b2,
           cm_W1, cm_b1, cm_W2, cm_b2, c1_W1, c1_b1, c1_W2, c1_b2, c2_W1,
           c2_b1, c2_W2, c2_b2, c3_W1, c3_b1, c3_W2, c3_b2, fc1_W, fc1_b,
           fc2_W, fc2_b, fc3_W, fc3_b):
    f32 = jnp.float32
    r1 = lambda b: b.reshape(1, -1)

    x0, pd, ps = _pre_call(
        var_node_features, con_node_features,
        vm_W1, r1(vm_b1), vm_W2, r1(vm_b2),
        cm_W1, r1(cm_b1), cm_W2, r1(cm_b2),
        c1_W1[:, :D], r1(c1_b1), c1_W1[:, D:2 * D])

    pad = NE_P - NE
    srcp = jnp.concatenate([edge_index[0],
                            jnp.full((pad, ), NN, jnp.int32)])
    dstp = jnp.concatenate([edge_index[1],
                            jnp.full((pad, ), NN, jnp.int32)])
    etw = jnp.concatenate([edge_types,
                           jnp.zeros((pad, 2), f32)]).reshape(NE_P // 2, 4)

    def w4(W1):
        we = W1[:, 2 * D:]
        z = jnp.zeros((2 * D, 4), f32)
        return z.at[:D, :2].set(we).at[D:, 2:].set(we)

    cntv = _sc_cnt()(dstp)
    sc = _sc_edge()
    s1 = sc(pd, ps, srcp, dstp, _pe_call(etw, w4(c1_W1)))
    x1, pd, ps = _mid_call(s1, cntv, c1_W2, r1(c1_b2),
                           c2_W1[:, :D], r1(c2_b1), c2_W1[:, D:2 * D])
    s2 = sc(pd, ps, srcp, dstp, _pe_call(etw, w4(c2_W1)))
    x2, pd, ps = _mid_call(s2, cntv, c2_W2, r1(c2_b2),
                           c3_W1[:, :D], r1(c3_b1), c3_W1[:, D:2 * D])
    s3 = sc(pd, ps, srcp, dstp, _pe_call(etw, w4(c3_W1)))

    out = _fin_call(s3, cntv, x0, x1, x2, c3_W2, r1(c3_b2),
                    fc1_W[:, :D], fc1_W[:, D:2 * D],
                    fc1_W[:, 2 * D:3 * D], fc1_W[:, 3 * D:],
                    r1(fc1_b), fc2_W, r1(fc2_b),
                    jnp.zeros((128, D), f32).at[0].set(fc3_W[0]),
                    jnp.zeros((1, 128), f32).at[0, 0].set(fc3_b[0]))
    return out[:, 0]


# speedup vs baseline: 1.5093x; 1.0587x over previous
import functools

import jax
import jax.numpy as jnp
from jax import lax
from jax.experimental import pallas as pl
from jax.experimental.pallas import tpu as pltpu
from jax.experimental.pallas import tpu_sc as plsc

NN = 50000
NV = 25000
D = 64
NE = 800000
NT = 16
CH = 64
CPT = 782
NE_P = NT * CPT * CH
CW = 16
SROWS = 25088
JUNK = NV
R = 1000
WR = 200


def _iprep_chunk(dstv, srcv, gdi, gsi, li, off):
    def _iprep(q, _):
        sl = pl.ds(q * 16, 16)
        dd = dstv[sl]
        if gdi is not None:
            gdi[sl] = jnp.minimum(dd, NN - 1)
        if srcv is not None:
            gsi[sl] = jnp.minimum(srcv[sl], NN - 1)
        ll = dd - off
        ok = (ll >= 0) & (ll < NV)
        li[sl] = jnp.where(ok, ll, JUNK)
        return 0
    lax.fori_loop(0, CH // 16, _iprep, 0)


def _zero_shared(s, buf, s_sh, rows, width):
    zero16 = jnp.zeros((16,), jnp.float32)

    def _zrow(i, _):
        for q in range(width // 16):
            buf[i, pl.ds(q * 16, 16)] = zero16
        return 0
    lax.fori_loop(0, rows, _zrow, 0)

    def _zchunk(k, _):
        j = s + k * NT

        @pl.when(j < SROWS // rows)
        def _():
            pltpu.sync_copy(buf, s_sh.at[pl.ds(j * rows, rows)])
        return 0
    lax.fori_loop(0, (SROWS // rows + NT - 1) // NT, _zchunk, 0)


def _writeback(s, s_sh, out_hbm, off):
    def _wb(k, _):
        j = s + k * NT

        @pl.when(j < NV // WR)
        def _():
            pltpu.sync_copy(s_sh.at[pl.ds(j * WR, WR)],
                            out_hbm.at[pl.ds(off + j * WR, WR)])
        return 0
    lax.fori_loop(0, (NV // WR + NT - 1) // NT, _wb, 0)


def _sc_edge_body(pd_hbm, ps_hbm, src_hbm, dst_hbm, pe_hbm, out_hbm,
                  dstv0, srcv0, gdi0, gsi0, li0, pev0, gdv0, gsv0,
                  dstv1, srcv1, gdi1, gsi1, li1, pev1, gdv1, gsv1,
                  s_sh, sem_in0, sem_in1, sem_sc0, sem_sc1):
    c = lax.axis_index("c")
    s = lax.axis_index("s")
    off = c * NV

    _zero_shared(s, gdv0, s_sh, CH, D)
    plsc.subcore_barrier()

    ebase = s * (CPT * CH)
    bufs = ((dstv0, srcv0, gdi0, gsi0, li0, pev0, gdv0, gsv0,
             sem_in0, sem_sc0),
            (dstv1, srcv1, gdi1, gsi1, li1, pev1, gdv1, gsv1,
             sem_in1, sem_sc1))

    def _prep(kn, buf):
        dstv, srcv, gdi, gsi, li, pev, gdv, gsv, sem_in, _ = buf
        b = ebase + kn * CH
        pltpu.sync_copy(dst_hbm.at[pl.ds(b, CH)], dstv)
        pltpu.sync_copy(src_hbm.at[pl.ds(b, CH)], srcv)
        _iprep_chunk(dstv, srcv, gdi, gsi, li, off)
        pltpu.async_copy(pe_hbm.at[pl.ds(b // 2, CH // 2)], pev, sem_in)
        pltpu.async_copy(pd_hbm.at[gdi], gdv, sem_in)
        pltpu.async_copy(ps_hbm.at[gsi], gsv, sem_in)

    def _wait_in(kn, buf):
        dstv, srcv, gdi, gsi, li, pev, gdv, gsv, sem_in, _ = buf
        b = ebase + kn * CH
        pltpu.make_async_copy(pe_hbm.at[pl.ds(b // 2, CH // 2)],
                              pev, sem_in).wait()
        pltpu.make_async_copy(pd_hbm.at[gdi], gdv, sem_in).wait()
        pltpu.make_async_copy(ps_hbm.at[gsi], gsv, sem_in).wait()

    def _compute(buf):
        pev, gdv, gsv = buf[5], buf[6], buf[7]

        def _pair(i2, _):
            for h in range(2):
                i = 2 * i2 + h
                for j in range(4):
                    sl = pl.ds(j * 16, 16)
                    m = (gdv[i, sl] + gsv[i, sl]
                         + pev[i2, pl.ds(h * 64 + j * 16, 16)])
                    gdv[i, sl] = jnp.maximum(m, 0.0)
            return 0
        lax.fori_loop(0, CH // 2, _pair, 0)

    def _fire_scatter(buf):
        li, gdv, sem_sc = buf[4], buf[6], buf[9]
        pltpu.async_copy(gdv, s_sh.at[li], sem_sc, add=True)

    def _wait_scatter(buf):
        li, gdv, sem_sc = buf[4], buf[6], buf[9]
        pltpu.make_async_copy(gdv, s_sh.at[li], sem_sc).wait()

    _prep(0, bufs[0])

    def _body(k2, _):
        k0 = 2 * k2

        @pl.when(k2 > 0)
        def _():
            _wait_scatter(bufs[1])
        _prep(k0 + 1, bufs[1])
        _wait_in(k0, bufs[0])
        _compute(bufs[0])
        _fire_scatter(bufs[0])
        _wait_scatter(bufs[0])

        @pl.when(k0 + 2 < CPT)
        def _():
            _prep(k0 + 2, bufs[0])
        _wait_in(k0 + 1, bufs[1])
        _compute(bufs[1])
        _fire_scatter(bufs[1])
        return 0
    lax.fori_loop(0, CPT // 2, _body, 0)
    _wait_scatter(bufs[1])
    plsc.subcore_barrier()
    _writeback(s, s_sh, out_hbm, off)


def _sc_cnt_body(dst_hbm, out_hbm, dstv0, li0, dstv1, li1, onev, s_sh,
                 sem_d0, sem_d1, sem_sc0, sem_sc1):
    c = lax.axis_index("c")
    s = lax.axis_index("s")
    off = c * NV

    _zero_shared(s, onev, s_sh, CH, CW)
    plsc.subcore_barrier()

    lane = lax.iota(jnp.int32, 16)
    onecol = jnp.where(lane == 0, 1.0, 0.0).astype(jnp.float32)

    def _orow(i, _):
        onev[i, pl.ds(0, 16)] = onecol
        return 0
    lax.fori_loop(0, CH, _orow, 0)

    ebase = s * (CPT * CH)
    bufs = ((dstv0, li0, sem_d0, sem_sc0), (dstv1, li1, sem_d1, sem_sc1))

    def _stage(kn, buf):
        dstv, li, sem_d, _ = buf
        pltpu.async_copy(dst_hbm.at[pl.ds(ebase + kn * CH, CH)], dstv, sem_d)

    def _scat(kn, buf):
        dstv, li, sem_d, sem_sc = buf
        pltpu.make_async_copy(dst_hbm.at[pl.ds(ebase + kn * CH, CH)],
                             dstv, sem_d).wait()
        _iprep_chunk(dstv, None, None, None, li, off)
        pltpu.async_copy(onev, s_sh.at[li], sem_sc, add=True)

    def _wait_sc(buf):
        dstv, li, sem_d, sem_sc = buf
        pltpu.make_async_copy(onev, s_sh.at[li], sem_sc).wait()

    _stage(0, bufs[0])

    def _body(k2, _):
        k0 = 2 * k2
        _stage(k0 + 1, bufs[1])

        @pl.when(k2 > 0)
        def _():
            _wait_sc(bufs[0])
        _scat(k0, bufs[0])

        @pl.when(k0 + 2 < CPT)
        def _():
            _stage(k0 + 2, bufs[0])

        @pl.when(k2 > 0)
        def _():
            _wait_sc(bufs[1])
        _scat(k0 + 1, bufs[1])
        return 0
    lax.fori_loop(0, CPT // 2, _body, 0)
    _wait_sc(bufs[0])
    _wait_sc(bufs[1])
    plsc.subcore_barrier()
    _writeback(s, s_sh, out_hbm, off)


_SC_MESH = dict(core_axis_name="c", subcore_axis_name="s",
                num_cores=2, num_subcores=NT)


@functools.cache
def _sc_edge():
    return pl.kernel(
        _sc_edge_body,
        out_type=jax.ShapeDtypeStruct((NN, D), jnp.float32),
        mesh=plsc.VectorSubcoreMesh(**_SC_MESH),
        scratch_types=(
            [pltpu.VMEM((CH,), jnp.int32)] * 5
            + [pltpu.VMEM((CH // 2, 2 * D), jnp.float32),
               pltpu.VMEM((CH, D), jnp.float32),
               pltpu.VMEM((CH, D), jnp.float32)]
        ) * 2 + [
            pltpu.VMEM_SHARED((SROWS, D), jnp.float32),
            pltpu.SemaphoreType.DMA,
            pltpu.SemaphoreType.DMA,
            pltpu.SemaphoreType.DMA,
            pltpu.SemaphoreType.DMA,
        ],
        compiler_params=pltpu.CompilerParams(use_tc_tiling_on_sc=False),
    )


@functools.cache
def _sc_cnt():
    return pl.kernel(
        _sc_cnt_body,
        out_type=jax.ShapeDtypeStruct((NN, CW), jnp.float32),
        mesh=plsc.VectorSubcoreMesh(**_SC_MESH),
        scratch_types=[
            pltpu.VMEM((CH,), jnp.int32),
            pltpu.VMEM((CH,), jnp.int32),
            pltpu.VMEM((CH,), jnp.int32),
            pltpu.VMEM((CH,), jnp.int32),
            pltpu.VMEM((CH, CW), jnp.float32),
            pltpu.VMEM_SHARED((SROWS, CW), jnp.float32),
            pltpu.SemaphoreType.DMA,
            pltpu.SemaphoreType.DMA,
            pltpu.SemaphoreType.DMA,
            pltpu.SemaphoreType.DMA,
        ],
        compiler_params=pltpu.CompilerParams(use_tc_tiling_on_sc=False,
                                             needs_layout_passes=False),
    )


_PERM = sum(([q * 32 + 2 * k for k in range(16)]
             + [q * 32 + 2 * k + 1 for k in range(16)]
             for q in range(2)), [])


def _pe_body(et, w, peo):
    peo[...] = et[...].astype(jnp.float32) @ w[...].T


RE = 25024

_pe_call = pl.pallas_call(
    _pe_body,
    grid=(NE_P // 2 // RE,),
    in_specs=[pl.BlockSpec((RE, 4), lambda i: (i, 0)),
              pl.BlockSpec((2 * D, 4), lambda i: (0, 0))],
    out_specs=pl.BlockSpec((RE, 2 * D), lambda i: (i, 0)),
    out_shape=jax.ShapeDtypeStruct((NE_P // 2, 2 * D), jnp.float32),
)


def _pre_body(vf, cf, vW1, vb1, vW2, vb2, cW1, cb1, cW2, cb2,
              W1d, nb1, W1s, x0o, pdo, pso):
    isv = pl.program_id(0) < NV // R
    f = jnp.where(isv, vf[...], cf[...])
    W1 = jnp.where(isv, vW1[...], cW1[...])
    b1 = jnp.where(isv, vb1[...], cb1[...])
    W2 = jnp.where(isv, vW2[...], cW2[...])
    b2 = jnp.where(isv, vb2[...], cb2[...])
    x = jax.nn.relu(f @ W1.T + b1) @ W2.T + b2
    x0o[...] = x
    pdo[...] = x @ W1d[...].T + nb1[...]
    pso[...] = x @ W1s[...].T


def _mid_body(S, C, W2, b2, W1d, nb1, W1s, xo, pdo, pso):
    cnt = C[...][:, :1]
    x = jax.nn.relu((S[...] @ W2[...].T + cnt * b2[...])
                    / jnp.maximum(cnt, 1.0))
    xo[...] = x
    pdo[...] = x @ W1d[...].T + nb1[...]
    pso[...] = x @ W1s[...].T


def _fin_body(S3, C, x0, x1, x2, W2, b2, F0, F1, F2, F3,
              fb1, fW2, fb2, fW3, fb3, out):
    cnt = C[...][:, :1]
    x3 = jax.nn.relu((S3[...] @ W2[...].T + cnt * b2[...])
                     / jnp.maximum(cnt, 1.0))
    h = (x0[...] @ F0[...].T + x1[...] @ F1[...].T + x2[...] @ F2[...].T
         + x3 @ F3[...].T + fb1[...])
    h = jax.nn.relu(h)
    h = jax.nn.relu(h @ fW2[...].T + fb2[...])
    out[...] = jax.nn.sigmoid(h @ fW3[...].T + fb3[...])


def _full(shape):
    return pl.BlockSpec(shape, lambda i: tuple(0 for _ in shape))


def _rows(w):
    return pl.BlockSpec((R, w), lambda i: (i, 0))


_pre_call = pl.pallas_call(
    _pre_body,
    grid=(NN // R,),
    in_specs=[
        pl.BlockSpec((R, 2), lambda i: (jnp.minimum(i, NV // R - 1), 0)),
        pl.BlockSpec((R, 2), lambda i: (jnp.maximum(i - NV // R, 0), 0)),
        _full((D, 2)), _full((1, D)), _full((D, D)), _full((1, D)),
        _full((D, 2)), _full((1, D)), _full((D, D)), _full((1, D)),
        _full((D, D)), _full((1, D)), _full((D, D)),
    ],
    out_specs=[_rows(D), _rows(D), _rows(D)],
    out_shape=[jax.ShapeDtypeStruct((NN, D), jnp.float32)] * 3,
)

_mid_call = pl.pallas_call(
    _mid_body,
    grid=(NN // R,),
    in_specs=[
        _rows(D), _rows(CW),
        _full((D, D)), _full((1, D)),
        _full((D, D)), _full((1, D)), _full((D, D)),
    ],
    out_specs=[_rows(D), _rows(D), _rows(D)],
    out_shape=[jax.ShapeDtypeStruct((NN, D), jnp.float32)] * 3,
)

_fin_call = pl.pallas_call(
    _fin_body,
    grid=(NV // R,),
    in_specs=[
        _rows(D), _rows(CW), _rows(D), _rows(D), _rows(D),
        _full((D, D)), _full((1, D)),
        _full((D, D)), _full((D, D)), _full((D, D)), _full((D, D)),
        _full((1, D)), _full((D, D)), _full((1, D)),
        _full((128, D)), _full((1, 128)),
    ],
    out_specs=pl.BlockSpec((R, 128), lambda i: (i, 0)),
    out_shape=jax.ShapeDtypeStruct((NV, 128), jnp.float32),
)


def kernel(var_node_features, con_node_features, node_types, assoc_var,
           assoc_con, edge_index, edge_types, vm_W1, vm_b1, vm_W2, vm_b2,
           cm_W1, cm_b1, cm_W2, cm_b2, c1_W1, c1_b1, c1_W2, c1_b2, c2_W1,
           c2_b1, c2_W2, c2_b2, c3_W1, c3_b1, c3_W2, c3_b2, fc1_W, fc1_b,
           fc2_W, fc2_b, fc3_W, fc3_b):
    f32 = jnp.float32
    r1 = lambda b: b.reshape(1, -1)

    x0, pd, ps = _pre_call(
        var_node_features, con_node_features,
        vm_W1, r1(vm_b1), vm_W2, r1(vm_b2),
        cm_W1, r1(cm_b1), cm_W2, r1(cm_b2),
        c1_W1[:, :D], r1(c1_b1), c1_W1[:, D:2 * D])

    pad = NE_P - NE
    srcp = jnp.concatenate([edge_index[0],
                            jnp.full((pad, ), NN, jnp.int32)])
    dstp = jnp.concatenate([edge_index[1],
                            jnp.full((pad, ), NN, jnp.int32)])
    etw = jnp.concatenate([edge_types.astype(jnp.bfloat16),
                           jnp.zeros((pad, 2), jnp.bfloat16)]
                          ).reshape(NE_P // 2, 4)

    def w4(W1):
        we = W1[:, 2 * D:]
        z = jnp.zeros((2 * D, 4), f32)
        return z.at[:D, :2].set(we).at[D:, 2:].set(we)

    cntv = _sc_cnt()(dstp)
    sc = _sc_edge()
    s1 = sc(pd, ps, srcp, dstp, _pe_call(etw, w4(c1_W1)))
    x1, pd, ps = _mid_call(s1, cntv, c1_W2, r1(c1_b2),
                           c2_W1[:, :D], r1(c2_b1), c2_W1[:, D:2 * D])
    s2 = sc(pd, ps, srcp, dstp, _pe_call(etw, w4(c2_W1)))
    x2, pd, ps = _mid_call(s2, cntv, c2_W2, r1(c2_b2),
                           c3_W1[:, :D], r1(c3_b1), c3_W1[:, D:2 * D])
    s3 = sc(pd, ps, srcp, dstp, _pe_call(etw, w4(c3_W1)))

    out = _fin_call(s3, cntv, x0, x1, x2, c3_W2, r1(c3_b2),
                    fc1_W[:, :D], fc1_W[:, D:2 * D],
                    fc1_W[:, 2 * D:3 * D], fc1_W[:, 3 * D:],
                    r1(fc1_b), fc2_W, r1(fc2_b),
                    jnp.zeros((128, D), f32).at[0].set(fc3_W[0]),
                    jnp.zeros((1, 128), f32).at[0, 0].set(fc3_b[0]))
    return out[:, 0]


# speedup vs baseline: 1.7650x; 1.1694x over previous
import functools

import jax
import jax.numpy as jnp
from jax import lax
from jax.experimental import pallas as pl
from jax.experimental.pallas import tpu as pltpu
from jax.experimental.pallas import tpu_sc as plsc

NN = 50000
NV = 25000
D = 64
NE = 800000
NT = 16
CH = 64
CPT = 782
NE_P = NT * CPT * CH
CW = 16
SROWS = 25088
JUNK = NV
R = 1000
WR = 200


def _iprep_chunk(dstv, srcv, gdi, gsi, li, off):
    def _iprep(q, _):
        sl = pl.ds(q * 16, 16)
        dd = dstv[sl]
        if gdi is not None:
            gdi[sl] = jnp.minimum(dd, NN - 1)
        if srcv is not None:
            gsi[sl] = jnp.minimum(srcv[sl], NN - 1)
        ll = dd - off
        ok = (ll >= 0) & (ll < NV)
        li[sl] = jnp.where(ok, ll, JUNK)
        return 0
    lax.fori_loop(0, CH // 16, _iprep, 0)


def _zero_shared(s, buf, s_sh, rows, width):
    zero16 = jnp.zeros((16,), jnp.float32)

    def _zrow(i, _):
        for q in range(width // 16):
            buf[i, pl.ds(q * 16, 16)] = zero16
        return 0
    lax.fori_loop(0, rows, _zrow, 0)

    def _zchunk(k, _):
        j = s + k * NT

        @pl.when(j < SROWS // rows)
        def _():
            pltpu.sync_copy(buf, s_sh.at[pl.ds(j * rows, rows)])
        return 0
    lax.fori_loop(0, (SROWS // rows + NT - 1) // NT, _zchunk, 0)


def _writeback(s, s_sh, out_hbm, off):
    def _wb(k, _):
        j = s + k * NT

        @pl.when(j < NV // WR)
        def _():
            pltpu.sync_copy(s_sh.at[pl.ds(j * WR, WR)],
                            out_hbm.at[pl.ds(off + j * WR, WR)])
        return 0
    lax.fori_loop(0, (NV // WR + NT - 1) // NT, _wb, 0)


def _sc_edge_body(pd_hbm, ps_hbm, ids_hbm, pe_hbm, out_hbm,
                  idsv0, gdi0, gsi0, li0, pev0, gdv0, gsv0,
                  idsv1, gdi1, gsi1, li1, pev1, gdv1, gsv1,
                  s_sh, sem_in0, sem_in1, sem_sc0, sem_sc1):
    c = lax.axis_index("c")
    s = lax.axis_index("s")
    off = c * NV

    _zero_shared(s, gdv0, s_sh, CH, D)
    plsc.subcore_barrier()

    ebase = s * (CPT * CH)
    bufs = ((idsv0, gdi0, gsi0, li0, pev0, gdv0, gsv0,
             sem_in0, sem_sc0),
            (idsv1, gdi1, gsi1, li1, pev1, gdv1, gsv1,
             sem_in1, sem_sc1))

    def _prep(kn, buf):
        idsv, gdi, gsi, li, pev, gdv, gsv, sem_in, _ = buf
        kc = ebase // CH + kn
        pltpu.sync_copy(ids_hbm.at[kc], idsv)

        def _iprep(q, _):
            sl = pl.ds(q * 16, 16)
            dd = idsv[sl]
            ss = idsv[pl.ds(CH + q * 16, 16)]
            gdi[sl] = jnp.minimum(dd, NN - 1)
            gsi[sl] = jnp.minimum(ss, NN - 1)
            ll = dd - off
            ok = (ll >= 0) & (ll < NV)
            li[sl] = jnp.where(ok, ll, JUNK)
            return 0
        lax.fori_loop(0, CH // 16, _iprep, 0)
        pltpu.async_copy(pe_hbm.at[pl.ds((ebase + kn * CH) // 2, CH // 2)],
                         pev, sem_in)
        pltpu.async_copy(pd_hbm.at[gdi], gdv, sem_in)
        pltpu.async_copy(ps_hbm.at[gsi], gsv, sem_in)

    def _wait_in(kn, buf):
        idsv, gdi, gsi, li, pev, gdv, gsv, sem_in, _ = buf
        b = ebase + kn * CH
        pltpu.make_async_copy(pe_hbm.at[pl.ds(b // 2, CH // 2)],
                              pev, sem_in).wait()
        pltpu.make_async_copy(pd_hbm.at[gdi], gdv, sem_in).wait()
        pltpu.make_async_copy(ps_hbm.at[gsi], gsv, sem_in).wait()

    def _compute(buf):
        pev, gdv, gsv = buf[4], buf[5], buf[6]

        def _pair(i2, _):
            for h in range(2):
                i = 2 * i2 + h
                for j in range(4):
                    sl = pl.ds(j * 16, 16)
                    m = (gdv[i, sl] + gsv[i, sl]
                         + pev[i2, pl.ds(h * 64 + j * 16, 16)])
                    gdv[i, sl] = jnp.maximum(m, 0.0)
            return 0
        lax.fori_loop(0, CH // 2, _pair, 0)

    def _fire_scatter(buf):
        li, gdv, sem_sc = buf[3], buf[5], buf[8]
        pltpu.async_copy(gdv, s_sh.at[li], sem_sc, add=True)

    def _wait_scatter(buf):
        li, gdv, sem_sc = buf[3], buf[5], buf[8]
        pltpu.make_async_copy(gdv, s_sh.at[li], sem_sc).wait()

    _prep(0, bufs[0])

    def _body(k2, _):
        k0 = 2 * k2

        @pl.when(k2 > 0)
        def _():
            _wait_scatter(bufs[1])
        _prep(k0 + 1, bufs[1])
        _wait_in(k0, bufs[0])
        _compute(bufs[0])
        _fire_scatter(bufs[0])
        _wait_scatter(bufs[0])

        @pl.when(k0 + 2 < CPT)
        def _():
            _prep(k0 + 2, bufs[0])
        _wait_in(k0 + 1, bufs[1])
        _compute(bufs[1])
        _fire_scatter(bufs[1])
        return 0
    lax.fori_loop(0, CPT // 2, _body, 0)
    _wait_scatter(bufs[1])
    plsc.subcore_barrier()
    _writeback(s, s_sh, out_hbm, off)


def _sc_cnt_body(dst_hbm, out_hbm, dstv0, li0, dstv1, li1, onev, s_sh,
                 sem_d0, sem_d1, sem_sc0, sem_sc1):
    c = lax.axis_index("c")
    s = lax.axis_index("s")
    off = c * NV

    _zero_shared(s, onev, s_sh, CH, CW)
    plsc.subcore_barrier()

    lane = lax.iota(jnp.int32, 16)
    onecol = jnp.where(lane == 0, 1.0, 0.0).astype(jnp.float32)

    def _orow(i, _):
        onev[i, pl.ds(0, 16)] = onecol
        return 0
    lax.fori_loop(0, CH, _orow, 0)

    ebase = s * (CPT * CH)
    bufs = ((dstv0, li0, sem_d0, sem_sc0), (dstv1, li1, sem_d1, sem_sc1))

    def _stage(kn, buf):
        dstv, li, sem_d, _ = buf
        pltpu.async_copy(dst_hbm.at[pl.ds(ebase + kn * CH, CH)], dstv, sem_d)

    def _scat(kn, buf):
        dstv, li, sem_d, sem_sc = buf
        pltpu.make_async_copy(dst_hbm.at[pl.ds(ebase + kn * CH, CH)],
                             dstv, sem_d).wait()
        _iprep_chunk(dstv, None, None, None, li, off)
        pltpu.async_copy(onev, s_sh.at[li], sem_sc, add=True)

    def _wait_sc(buf):
        dstv, li, sem_d, sem_sc = buf
        pltpu.make_async_copy(onev, s_sh.at[li], sem_sc).wait()

    _stage(0, bufs[0])

    def _body(k2, _):
        k0 = 2 * k2
        _stage(k0 + 1, bufs[1])

        @pl.when(k2 > 0)
        def _():
            _wait_sc(bufs[0])
        _scat(k0, bufs[0])

        @pl.when(k0 + 2 < CPT)
        def _():
            _stage(k0 + 2, bufs[0])

        @pl.when(k2 > 0)
        def _():
            _wait_sc(bufs[1])
        _scat(k0 + 1, bufs[1])
        return 0
    lax.fori_loop(0, CPT // 2, _body, 0)
    _wait_sc(bufs[0])
    _wait_sc(bufs[1])
    plsc.subcore_barrier()
    _writeback(s, s_sh, out_hbm, off)


_SC_MESH = dict(core_axis_name="c", subcore_axis_name="s",
                num_cores=2, num_subcores=NT)


@functools.cache
def _sc_edge():
    return pl.kernel(
        _sc_edge_body,
        out_type=jax.ShapeDtypeStruct((NN, D), jnp.float32),
        mesh=plsc.VectorSubcoreMesh(**_SC_MESH),
        scratch_types=(
            [pltpu.VMEM((2 * CH,), jnp.int32)]
            + [pltpu.VMEM((CH,), jnp.int32)] * 3
            + [pltpu.VMEM((CH // 2, 2 * D), jnp.float32),
               pltpu.VMEM((CH, D), jnp.float32),
               pltpu.VMEM((CH, D), jnp.float32)]
        ) * 2 + [
            pltpu.VMEM_SHARED((SROWS, D), jnp.float32),
            pltpu.SemaphoreType.DMA,
            pltpu.SemaphoreType.DMA,
            pltpu.SemaphoreType.DMA,
            pltpu.SemaphoreType.DMA,
        ],
        compiler_params=pltpu.CompilerParams(use_tc_tiling_on_sc=False),
    )


@functools.cache
def _sc_cnt():
    return pl.kernel(
        _sc_cnt_body,
        out_type=jax.ShapeDtypeStruct((NN, CW), jnp.float32),
        mesh=plsc.VectorSubcoreMesh(**_SC_MESH),
        scratch_types=[
            pltpu.VMEM((CH,), jnp.int32),
            pltpu.VMEM((CH,), jnp.int32),
            pltpu.VMEM((CH,), jnp.int32),
            pltpu.VMEM((CH,), jnp.int32),
            pltpu.VMEM((CH, CW), jnp.float32),
            pltpu.VMEM_SHARED((SROWS, CW), jnp.float32),
            pltpu.SemaphoreType.DMA,
            pltpu.SemaphoreType.DMA,
            pltpu.SemaphoreType.DMA,
            pltpu.SemaphoreType.DMA,
        ],
        compiler_params=pltpu.CompilerParams(use_tc_tiling_on_sc=False,
                                             needs_layout_passes=False),
    )


_PERM = sum(([q * 32 + 2 * k for k in range(16)]
             + [q * 32 + 2 * k + 1 for k in range(16)]
             for q in range(2)), [])


def _pe_body(et, w, peo):
    peo[...] = et[...].astype(jnp.float32) @ w[...].T


RE = 25024

_pe_call = pl.pallas_call(
    _pe_body,
    grid=(NE_P // 2 // RE,),
    in_specs=[pl.BlockSpec((RE, 4), lambda i: (i, 0)),
              pl.BlockSpec((2 * D, 4), lambda i: (0, 0))],
    out_specs=pl.BlockSpec((RE, 2 * D), lambda i: (i, 0)),
    out_shape=jax.ShapeDtypeStruct((NE_P // 2, 2 * D), jnp.float32),
)


def _pre_body(vf, cf, vW1, vb1, vW2, vb2, cW1, cb1, cW2, cb2,
              W1d, nb1, W1s, x0o, pdo, pso):
    isv = pl.program_id(0) < NV // R
    f = jnp.where(isv, vf[...], cf[...])
    W1 = jnp.where(isv, vW1[...], cW1[...])
    b1 = jnp.where(isv, vb1[...], cb1[...])
    W2 = jnp.where(isv, vW2[...], cW2[...])
    b2 = jnp.where(isv, vb2[...], cb2[...])
    x = jax.nn.relu(f @ W1.T + b1) @ W2.T + b2
    x0o[...] = x
    pdo[...] = x @ W1d[...].T + nb1[...]
    pso[...] = x @ W1s[...].T


def _mid_body(S, C, W2, b2, W1d, nb1, W1s, xo, pdo, pso):
    cnt = C[...][:, :1]
    x = jax.nn.relu((S[...] @ W2[...].T + cnt * b2[...])
                    / jnp.maximum(cnt, 1.0))
    xo[...] = x
    pdo[...] = x @ W1d[...].T + nb1[...]
    pso[...] = x @ W1s[...].T


def _fin_body(S3, C, x0, x1, x2, W2, b2, F0, F1, F2, F3,
              fb1, fW2, fb2, fW3, fb3, out):
    cnt = C[...][:, :1]
    x3 = jax.nn.relu((S3[...] @ W2[...].T + cnt * b2[...])
                     / jnp.maximum(cnt, 1.0))
    h = (x0[...] @ F0[...].T + x1[...] @ F1[...].T + x2[...] @ F2[...].T
         + x3 @ F3[...].T + fb1[...])
    h = jax.nn.relu(h)
    h = jax.nn.relu(h @ fW2[...].T + fb2[...])
    out[...] = jax.nn.sigmoid(h @ fW3[...].T + fb3[...])


def _full(shape):
    return pl.BlockSpec(shape, lambda i: tuple(0 for _ in shape))


def _rows(w):
    return pl.BlockSpec((R, w), lambda i: (i, 0))


_pre_call = pl.pallas_call(
    _pre_body,
    grid=(NN // R,),
    in_specs=[
        pl.BlockSpec((R, 2), lambda i: (jnp.minimum(i, NV // R - 1), 0)),
        pl.BlockSpec((R, 2), lambda i: (jnp.maximum(i - NV // R, 0), 0)),
        _full((D, 2)), _full((1, D)), _full((D, D)), _full((1, D)),
        _full((D, 2)), _full((1, D)), _full((D, D)), _full((1, D)),
        _full((D, D)), _full((1, D)), _full((D, D)),
    ],
    out_specs=[_rows(D), _rows(D), _rows(D)],
    out_shape=[jax.ShapeDtypeStruct((NN, D), jnp.float32)] * 3,
)

_mid_call = pl.pallas_call(
    _mid_body,
    grid=(NN // R,),
    in_specs=[
        _rows(D), _rows(CW),
        _full((D, D)), _full((1, D)),
        _full((D, D)), _full((1, D)), _full((D, D)),
    ],
    out_specs=[_rows(D), _rows(D), _rows(D)],
    out_shape=[jax.ShapeDtypeStruct((NN, D), jnp.float32)] * 3,
)

_fin_call = pl.pallas_call(
    _fin_body,
    grid=(NV // R,),
    in_specs=[
        _rows(D), _rows(CW), _rows(D), _rows(D), _rows(D),
        _full((D, D)), _full((1, D)),
        _full((D, D)), _full((D, D)), _full((D, D)), _full((D, D)),
        _full((1, D)), _full((D, D)), _full((1, D)),
        _full((128, D)), _full((1, 128)),
    ],
    out_specs=pl.BlockSpec((R, 128), lambda i: (i, 0)),
    out_shape=jax.ShapeDtypeStruct((NV, 128), jnp.float32),
)


def kernel(var_node_features, con_node_features, node_types, assoc_var,
           assoc_con, edge_index, edge_types, vm_W1, vm_b1, vm_W2, vm_b2,
           cm_W1, cm_b1, cm_W2, cm_b2, c1_W1, c1_b1, c1_W2, c1_b2, c2_W1,
           c2_b1, c2_W2, c2_b2, c3_W1, c3_b1, c3_W2, c3_b2, fc1_W, fc1_b,
           fc2_W, fc2_b, fc3_W, fc3_b):
    f32 = jnp.float32
    r1 = lambda b: b.reshape(1, -1)

    x0, pd, ps = _pre_call(
        var_node_features, con_node_features,
        vm_W1, r1(vm_b1), vm_W2, r1(vm_b2),
        cm_W1, r1(cm_b1), cm_W2, r1(cm_b2),
        c1_W1[:, :D], r1(c1_b1), c1_W1[:, D:2 * D])

    pad = NE_P - NE
    srcp = jnp.concatenate([edge_index[0],
                            jnp.full((pad, ), NN, jnp.int32)])
    dstp = jnp.concatenate([edge_index[1],
                            jnp.full((pad, ), NN, jnp.int32)])
    etw = jnp.concatenate([edge_types.astype(jnp.bfloat16),
                           jnp.zeros((pad, 2), jnp.bfloat16)]
                          ).reshape(NE_P // 2, 4)

    def w4(W1):
        we = W1[:, 2 * D:]
        z = jnp.zeros((2 * D, 4), f32)
        return z.at[:D, :2].set(we).at[D:, 2:].set(we)

    idsw = jnp.concatenate([dstp.reshape(-1, CH), srcp.reshape(-1, CH)],
                           axis=1)

    cntv = _sc_cnt()(dstp)
    sc = _sc_edge()
    s1 = sc(pd, ps, idsw, _pe_call(etw, w4(c1_W1)))
    x1, pd, ps = _mid_call(s1, cntv, c1_W2, r1(c1_b2),
                           c2_W1[:, :D], r1(c2_b1), c2_W1[:, D:2 * D])
    s2 = sc(pd, ps, idsw, _pe_call(etw, w4(c2_W1)))
    x2, pd, ps = _mid_call(s2, cntv, c2_W2, r1(c2_b2),
                           c3_W1[:, :D], r1(c3_b1), c3_W1[:, D:2 * D])
    s3 = sc(pd, ps, idsw, _pe_call(etw, w4(c3_W1)))

    out = _fin_call(s3, cntv, x0, x1, x2, c3_W2, r1(c3_b2),
                    fc1_W[:, :D], fc1_W[:, D:2 * D],
                    fc1_W[:, 2 * D:3 * D], fc1_W[:, 3 * D:],
                    r1(fc1_b), fc2_W, r1(fc2_b),
                    jnp.zeros((128, D), f32).at[0].set(fc3_W[0]),
                    jnp.zeros((1, 128), f32).at[0, 0].set(fc3_b[0]))
    return out[:, 0]
